# baseline (speedup 1.0000x reference)
import functools
import math

import jax
import jax.numpy as jnp
from jax import lax
from jax.experimental import pallas as pl
from jax.experimental.pallas import tpu as pltpu
from jax.experimental.pallas import tpu_sc as plsc

D_MODEL = 64
SCALE = math.sqrt(D_MODEL)
LANES = 16

NUM_CORES = 2
NUM_SUBCORES = 16
NW = NUM_CORES * NUM_SUBCORES

B_TOTAL = 4096 * 200
B_PER_W = B_TOTAL // NW
CHUNK = 128
N_CHUNKS = B_PER_W // CHUNK


@functools.partial(
    pl.kernel,
    out_type=jax.ShapeDtypeStruct((B_TOTAL, D_MODEL), jnp.float32),
    mesh=plsc.VectorSubcoreMesh(core_axis_name="c", subcore_axis_name="s"),
    compiler_params=pltpu.CompilerParams(use_tc_tiling_on_sc=False),
    scratch_types=[
        pltpu.VMEM((N_CHUNKS, CHUNK), jnp.int32),
        pltpu.VMEM((CHUNK, D_MODEL), jnp.float32),
        pltpu.SemaphoreType.DMA,
    ],
)
def _embed(table_hbm, idx_hbm, out_hbm, idx_v, rows_v, sem):
    wid = lax.axis_index("s") * NUM_CORES + lax.axis_index("c")
    base = wid * B_PER_W

    pltpu.sync_copy(idx_hbm.at[wid], idx_v)

    def chunk_body(c, carry):
        pltpu.async_copy(table_hbm.at[idx_v.at[c]], rows_v, sem).wait()

        def row_body(i, c2):
            for j in range(D_MODEL // LANES):
                sl = pl.ds(j * LANES, LANES)
                rows_v[i, sl] = rows_v[i, sl] * SCALE
            return c2

        lax.fori_loop(0, CHUNK, row_body, 0)

        pltpu.sync_copy(rows_v, out_hbm.at[pl.ds(base + c * CHUNK, CHUNK)])
        return carry

    lax.fori_loop(0, N_CHUNKS, chunk_body, 0)


def kernel(data, embedding_weight):
    idx = data.reshape(NW, N_CHUNKS, CHUNK).astype(jnp.int32)
    out = _embed(embedding_weight, idx)
    return out.reshape(data.shape + (D_MODEL,))


# speedup vs baseline: 1.2073x; 1.2073x over previous
import functools
import math

import jax
import jax.numpy as jnp
from jax import lax
from jax.experimental import pallas as pl
from jax.experimental.pallas import tpu as pltpu
from jax.experimental.pallas import tpu_sc as plsc

D_MODEL = 64
SCALE = math.sqrt(D_MODEL)
LANES = 16

NUM_CORES = 2
NUM_SUBCORES = 16
NW = NUM_CORES * NUM_SUBCORES

B_TOTAL = 4096 * 200
B_PER_W = B_TOTAL // NW
CHUNK = 128
N_CHUNKS = B_PER_W // CHUNK
NBUF = 4
N_GROUPS = N_CHUNKS // NBUF


@functools.partial(
    pl.kernel,
    out_type=jax.ShapeDtypeStruct((B_TOTAL, D_MODEL), jnp.float32),
    mesh=plsc.VectorSubcoreMesh(core_axis_name="c", subcore_axis_name="s"),
    compiler_params=pltpu.CompilerParams(use_tc_tiling_on_sc=False),
    scratch_types=(
        [pltpu.VMEM((N_CHUNKS, CHUNK), jnp.int32)]
        + [pltpu.VMEM((CHUNK, D_MODEL), jnp.float32) for _ in range(2 * NBUF)]
        + [pltpu.SemaphoreType.DMA for _ in range(2 * NBUF)]
    ),
)
def _embed(table_hbm, idx_hbm, out_hbm, idx_v, *bufs_and_sems):
    gbufs = bufs_and_sems[0:NBUF]
    sbufs = bufs_and_sems[NBUF:2 * NBUF]
    gsems = bufs_and_sems[2 * NBUF:3 * NBUF]
    ssems = bufs_and_sems[3 * NBUF:4 * NBUF]

    wid = lax.axis_index("s") * NUM_CORES + lax.axis_index("c")
    base = wid * B_PER_W

    pltpu.sync_copy(idx_hbm.at[wid], idx_v)

    for b in range(NBUF):
        pltpu.async_copy(table_hbm.at[idx_v.at[b]], gbufs[b], gsems[b])

    def group_body(g, carry):
        for b in range(NBUF):
            c = g * NBUF + b

            pltpu.make_async_copy(
                table_hbm.at[idx_v.at[c]], gbufs[b], gsems[b]
            ).wait()

            @pl.when(g > 0)
            def _():
                pltpu.make_async_copy(
                    sbufs[b],
                    out_hbm.at[pl.ds(base + (c - NBUF) * CHUNK, CHUNK)],
                    ssems[b],
                ).wait()

            def row4_body(r, c2):
                for rr in range(4):
                    i = r * 4 + rr
                    for j in range(D_MODEL // LANES):
                        sl = pl.ds(j * LANES, LANES)
                        sbufs[b][i, sl] = gbufs[b][i, sl] * SCALE
                return c2

            lax.fori_loop(0, CHUNK // 4, row4_body, 0)

            @pl.when(g < N_GROUPS - 1)
            def _():
                pltpu.async_copy(
                    table_hbm.at[idx_v.at[c + NBUF]], gbufs[b], gsems[b]
                )

            pltpu.async_copy(
                sbufs[b],
                out_hbm.at[pl.ds(base + c * CHUNK, CHUNK)],
                ssems[b],
            )
        return carry

    lax.fori_loop(0, N_GROUPS, group_body, 0)

    for b in range(NBUF):
        c = (N_GROUPS - 1) * NBUF + b
        pltpu.make_async_copy(
            sbufs[b],
            out_hbm.at[pl.ds(base + c * CHUNK, CHUNK)],
            ssems[b],
        ).wait()


def kernel(data, embedding_weight):
    idx = data.reshape(NW, N_CHUNKS, CHUNK).astype(jnp.int32)
    out = _embed(embedding_weight, idx)
    return out.reshape(data.shape + (D_MODEL,))
